# baseline (speedup 1.0000x reference)
import functools

import jax
import jax.numpy as jnp
from jax import lax
from jax.experimental import pallas as pl
from jax.experimental.pallas import tpu as pltpu
from jax.experimental.pallas import tpu_sc as plsc

N = 10000
E = 320000
D = 128

CHUNK = 80
NW = 32
CPW = 128
NCH_PAD = NW * CPW
E_PAD = NCH_PAD * CHUNK
NPAD = 10240
ROWS_PER_TILE = NPAD // 16
NSTG = 4
NGRP = CHUNK // 16


def _mm_body(x_ref, w_ref, b_ref, o_ref):
    o_ref[...] = lax.dot_general(
        x_ref[...], w_ref[...],
        dimension_numbers=(((1,), (1,)), ((), ())),
        preferred_element_type=jnp.float32,
    ) + b_ref[...]


def _linear(x, W, b):
    grid = 10
    blk = N // grid
    return pl.pallas_call(
        _mm_body,
        grid=(grid,),
        in_specs=[
            pl.BlockSpec((blk, D), lambda i: (i, 0)),
            pl.BlockSpec((D, D), lambda i: (0, 0)),
            pl.BlockSpec((1, D), lambda i: (0, 0)),
        ],
        out_specs=pl.BlockSpec((blk, D), lambda i: (i, 0)),
        out_shape=jax.ShapeDtypeStruct((N, D), jnp.float32),
    )(x, W, b.reshape(1, D))


def _add3_body(a_ref, b_ref, o_ref):
    o_ref[...] = a_ref[0] + b_ref[0]


def _combine(partials):
    grid = 10
    blk = N // grid
    return pl.pallas_call(
        _add3_body,
        grid=(grid,),
        in_specs=[
            pl.BlockSpec((1, blk, D), lambda i: (0, i, 0)),
            pl.BlockSpec((1, blk, D), lambda i: (1, i, 0)),
        ],
        out_specs=pl.BlockSpec((blk, D), lambda i: (i, 0)),
        out_shape=jax.ShapeDtypeStruct((N, D), jnp.float32),
    )(partials, partials)


def _sc_body(h_hbm, idx_hbm, ew_hbm, z_hbm, out_hbm,
             acc, rin0, rin1, rout0, rout1, stg, estg,
             gsem0, gsem1, ssem0, ssem1,
             isem0, isem1, isem2, isem3):
    c = lax.axis_index("c")
    s = lax.axis_index("s")
    w_id = s * 2 + c
    rins = (rin0, rin1)
    routs = (rout0, rout1)
    gsems = (gsem0, gsem1)
    ssems = (ssem0, ssem1)
    isems = (isem0, isem1, isem2, isem3)

    pltpu.sync_copy(z_hbm, acc.at[pl.ds(s * ROWS_PER_TILE, ROWS_PER_TILE)])

    def fire_stage(t, slot):
        ch = w_id + t * NW
        pltpu.async_copy(idx_hbm.at[ch], stg.at[slot], isems[slot])
        pltpu.async_copy(ew_hbm.at[ch], estg.at[slot], isems[slot])

    def wait_stage(slot):
        pltpu.make_async_copy(idx_hbm.at[0], stg.at[slot],
                              isems[slot]).wait()
        pltpu.make_async_copy(ew_hbm.at[0], estg.at[slot],
                              isems[slot]).wait()

    def fire_gather(stg_slot, row_slot):
        pltpu.async_copy(h_hbm.at[stg.at[stg_slot, 0]], rins[row_slot],
                         gsems[row_slot])

    def wait_gather(row_slot):
        pltpu.make_async_copy(h_hbm.at[stg.at[0, 0]], rins[row_slot],
                              gsems[row_slot]).wait()

    def fire_scatter(stg_slot, row_slot):
        pltpu.async_copy(routs[row_slot], acc.at[stg.at[stg_slot, 1]],
                         ssems[row_slot], add=True)

    def wait_scatter(row_slot):
        pltpu.make_async_copy(routs[row_slot], acc.at[stg.at[0, 1]],
                              ssems[row_slot]).wait()

    def scale(bin_, bout, stg_slot):
        def g_body(g, carry):
            w16 = estg[stg_slot, 0, pl.ds(g * 16, 16)]
            for i in range(16):
                wb = w16[i]
                r = g * 16 + i
                for cc in range(8):
                    bout[r, pl.ds(cc * 16, 16)] = (
                        bin_[r, pl.ds(cc * 16, 16)] * wb)
            return carry

        lax.fori_loop(0, NGRP, g_body, 0)

    fire_stage(0, 0)
    fire_stage(1, 1)
    plsc.subcore_barrier()
    wait_stage(0)
    fire_gather(0, 0)

    def outer(j4, carry):
        t0 = j4 * NSTG
        for b in range(NSTG):
            t = t0 + b
            rb = b % 2
            prb = (b + 1) % 2

            @pl.when(t + 1 < CPW)
            def _():
                wait_stage((b + 1) % NSTG)
                fire_gather((b + 1) % NSTG, prb)

            wait_gather(rb)

            @pl.when(t >= 2)
            def _():
                wait_scatter(rb)

            @pl.when(t + 2 < CPW)
            def _():
                fire_stage(t + 2, (b + 2) % NSTG)

            scale(rins[rb], routs[rb], b)
            fire_scatter(b, rb)
        return carry

    lax.fori_loop(0, CPW // NSTG, outer, 0)
    wait_scatter(0)
    wait_scatter(1)

    plsc.subcore_barrier()
    pltpu.sync_copy(acc.at[pl.ds(s * ROWS_PER_TILE, ROWS_PER_TILE)],
                    out_hbm.at[c, pl.ds(s * ROWS_PER_TILE, ROWS_PER_TILE)])


_sc_aggregate = functools.partial(
    pl.kernel,
    out_type=jax.ShapeDtypeStruct((2, NPAD, D), jnp.float32),
    mesh=plsc.VectorSubcoreMesh(core_axis_name="c", subcore_axis_name="s"),
    compiler_params=pltpu.CompilerParams(needs_layout_passes=False),
    scratch_types=[
        pltpu.VMEM_SHARED((NPAD, D), jnp.float32),
        pltpu.VMEM((CHUNK, D), jnp.float32),
        pltpu.VMEM((CHUNK, D), jnp.float32),
        pltpu.VMEM((CHUNK, D), jnp.float32),
        pltpu.VMEM((CHUNK, D), jnp.float32),
        pltpu.VMEM((NSTG, 2, CHUNK), jnp.int32),
        pltpu.VMEM((NSTG, 1, CHUNK), jnp.float32),
    ] + [pltpu.SemaphoreType.DMA] * 8,
)(_sc_body)


def kernel(x, edge_index, edge_weight, W, b):
    h = _linear(x, W, b)
    pad = E_PAD - E
    r = jnp.arange(pad, dtype=jnp.int32)
    src = jnp.concatenate([edge_index[0], r % N])
    dst = jnp.concatenate([edge_index[1], r % NPAD])
    ew = jnp.concatenate([edge_weight, jnp.zeros((pad,), jnp.float32)])
    idx = jnp.stack(
        [src.reshape(NCH_PAD, CHUNK), dst.reshape(NCH_PAD, CHUNK)], axis=1)
    zeros = jnp.zeros((ROWS_PER_TILE, D), jnp.float32)
    partials = _sc_aggregate(h, idx, ew.reshape(NCH_PAD, 1, CHUNK), zeros)
    return _combine(partials)


# speedup vs baseline: 1.0616x; 1.0616x over previous
import functools

import jax
import jax.numpy as jnp
from jax import lax
from jax.experimental import pallas as pl
from jax.experimental.pallas import tpu as pltpu
from jax.experimental.pallas import tpu_sc as plsc

N = 10000
E = 320000
D = 128

CHUNK = 128
NW = 32
CPW = 80
NCH_PAD = NW * CPW
E_PAD = NCH_PAD * CHUNK
NPAD = 10240
ROWS_PER_TILE = NPAD // 16
NSTG = 4


def _mm_body(x_ref, w_ref, b_ref, o_ref):
    o_ref[...] = lax.dot_general(
        x_ref[...], w_ref[...],
        dimension_numbers=(((1,), (1,)), ((), ())),
        preferred_element_type=jnp.float32,
    ) + b_ref[...]


def _linear(x, W, b):
    grid = 10
    blk = N // grid
    return pl.pallas_call(
        _mm_body,
        grid=(grid,),
        in_specs=[
            pl.BlockSpec((blk, D), lambda i: (i, 0)),
            pl.BlockSpec((D, D), lambda i: (0, 0)),
            pl.BlockSpec((1, D), lambda i: (0, 0)),
        ],
        out_specs=pl.BlockSpec((blk, D), lambda i: (i, 0)),
        out_shape=jax.ShapeDtypeStruct((N, D), jnp.float32),
    )(x, W, b.reshape(1, D))


def _add3_body(a_ref, b_ref, o_ref):
    o_ref[...] = a_ref[0] + b_ref[0]


def _combine(partials):
    grid = 10
    blk = N // grid
    return pl.pallas_call(
        _add3_body,
        grid=(grid,),
        in_specs=[
            pl.BlockSpec((1, blk, D), lambda i: (0, i, 0)),
            pl.BlockSpec((1, blk, D), lambda i: (1, i, 0)),
        ],
        out_specs=pl.BlockSpec((blk, D), lambda i: (i, 0)),
        out_shape=jax.ShapeDtypeStruct((N, D), jnp.float32),
    )(partials, partials)


def _sc_body(h_hbm, idx3_hbm, ew_hbm, z_hbm, out_hbm,
             acc, rows0, rows1, stg, estg,
             gsem0, gsem1, ssem0, ssem1,
             isem0, isem1, isem2, isem3):
    c = lax.axis_index("c")
    s = lax.axis_index("s")
    w_id = s * 2 + c
    rows = (rows0, rows1)
    gsems = (gsem0, gsem1)
    ssems = (ssem0, ssem1)
    isems = (isem0, isem1, isem2, isem3)

    pltpu.sync_copy(z_hbm, acc.at[pl.ds(s * ROWS_PER_TILE, ROWS_PER_TILE)])

    def fire_stage(t, slot):
        ch = w_id + t * NW
        pltpu.async_copy(idx3_hbm.at[ch], stg.at[slot], isems[slot])
        pltpu.async_copy(ew_hbm.at[ch], estg.at[slot], isems[slot])

    def wait_stage(slot):
        pltpu.make_async_copy(idx3_hbm.at[0], stg.at[slot],
                              isems[slot]).wait()
        pltpu.make_async_copy(ew_hbm.at[0], estg.at[slot],
                              isems[slot]).wait()

    def fire_gather(stg_slot, row_slot):
        pltpu.async_copy(h_hbm.at[stg.at[stg_slot, 0]], rows[row_slot],
                         gsems[row_slot])

    def wait_gather(row_slot):
        pltpu.make_async_copy(h_hbm.at[stg.at[0, 0]], rows[row_slot],
                              gsems[row_slot]).wait()

    def fire_scatter(stg_slot, row_slot):
        pltpu.async_copy(rows[row_slot], acc.at[stg.at[stg_slot, 1]],
                         ssems[row_slot], add=True)

    def wait_scatter(row_slot):
        pltpu.make_async_copy(rows[row_slot], acc.at[stg.at[0, 1]],
                              ssems[row_slot]).wait()

    def scale_half(buf, stg_slot, g_lo, g_hi):
        def g_body(g, carry):
            w16 = estg[stg_slot, 0, pl.ds(g * 16, 16)]
            for i in range(16):
                wb = w16[i]
                r = g * 16 + i
                for cc in range(8):
                    buf[r, pl.ds(cc * 16, 16)] = (
                        buf[r, pl.ds(cc * 16, 16)] * wb)
            return carry

        lax.fori_loop(g_lo, g_hi, g_body, 0)

    for t in range(NSTG - 1):
        fire_stage(t, t)
    plsc.subcore_barrier()
    wait_stage(0)
    fire_gather(0, 0)

    def outer(j4, carry):
        t0 = j4 * NSTG
        for b in range(NSTG):
            t = t0 + b
            rb = b % 2
            prb = (b + 1) % 2
            wait_gather(rb)
            scale_half(rows[rb], b, 0, 4)

            @pl.when(t >= 1)
            def _():
                wait_scatter(prb)

            @pl.when(t + NSTG - 1 < CPW)
            def _():
                fire_stage(t + NSTG - 1, (b + NSTG - 1) % NSTG)

            @pl.when(t + 1 < CPW)
            def _():
                wait_stage((b + 1) % NSTG)
                fire_gather((b + 1) % NSTG, prb)

            scale_half(rows[rb], b, 4, 8)
            fire_scatter(b, rb)
        return carry

    lax.fori_loop(0, CPW // NSTG, outer, 0)
    wait_scatter((CPW - 1) % 2)

    plsc.subcore_barrier()
    pltpu.sync_copy(acc.at[pl.ds(s * ROWS_PER_TILE, ROWS_PER_TILE)],
                    out_hbm.at[c, pl.ds(s * ROWS_PER_TILE, ROWS_PER_TILE)])


_sc_aggregate = functools.partial(
    pl.kernel,
    out_type=jax.ShapeDtypeStruct((2, NPAD, D), jnp.float32),
    mesh=plsc.VectorSubcoreMesh(core_axis_name="c", subcore_axis_name="s"),
    compiler_params=pltpu.CompilerParams(needs_layout_passes=False),
    scratch_types=[
        pltpu.VMEM_SHARED((NPAD, D), jnp.float32),
        pltpu.VMEM((CHUNK, D), jnp.float32),
        pltpu.VMEM((CHUNK, D), jnp.float32),
        pltpu.VMEM((NSTG, 2, CHUNK), jnp.int32),
        pltpu.VMEM((NSTG, 1, CHUNK), jnp.float32),
    ] + [pltpu.SemaphoreType.DMA] * 8,
)(_sc_body)


def kernel(x, edge_index, edge_weight, W, b):
    h = _linear(x, W, b)
    pad = E_PAD - E
    r = jnp.arange(pad, dtype=jnp.int32)
    src = jnp.concatenate([edge_index[0], r % N])
    dst = jnp.concatenate([edge_index[1], r % NPAD])
    ew = jnp.concatenate([edge_weight, jnp.zeros((pad,), jnp.float32)])
    idx3 = jnp.stack(
        [src.reshape(NCH_PAD, CHUNK), dst.reshape(NCH_PAD, CHUNK)], axis=1)
    zeros = jnp.zeros((ROWS_PER_TILE, D), jnp.float32)
    partials = _sc_aggregate(h, idx3, ew.reshape(NCH_PAD, 1, CHUNK), zeros)
    return _combine(partials)


# speedup vs baseline: 1.1379x; 1.0719x over previous
import functools

import jax
import jax.numpy as jnp
from jax import lax
from jax.experimental import pallas as pl
from jax.experimental.pallas import tpu as pltpu
from jax.experimental.pallas import tpu_sc as plsc

N = 10000
E = 320000
D = 128

CHUNK = 128
NW = 32
NCH = E // CHUNK
MAXC = 80
NPAD = 10240
ROWS_PER_TILE = NPAD // 16
NSTG = 4


def _mm_body(x_ref, w_ref, b_ref, o_ref):
    o_ref[...] = lax.dot_general(
        x_ref[...], w_ref[...],
        dimension_numbers=(((1,), (1,)), ((), ())),
        preferred_element_type=jnp.float32,
    ) + b_ref[...]


def _linear(x, W, b):
    grid = 10
    blk = N // grid
    return pl.pallas_call(
        _mm_body,
        grid=(grid,),
        in_specs=[
            pl.BlockSpec((blk, D), lambda i: (i, 0)),
            pl.BlockSpec((D, D), lambda i: (0, 0)),
            pl.BlockSpec((1, D), lambda i: (0, 0)),
        ],
        out_specs=pl.BlockSpec((blk, D), lambda i: (i, 0)),
        out_shape=jax.ShapeDtypeStruct((N, D), jnp.float32),
    )(x, W, b.reshape(1, D))


def _add3_body(a_ref, b_ref, o_ref):
    o_ref[...] = a_ref[0] + b_ref[0]


def _combine(partials):
    grid = 10
    blk = N // grid
    return pl.pallas_call(
        _add3_body,
        grid=(grid,),
        in_specs=[
            pl.BlockSpec((1, blk, D), lambda i: (0, i, 0)),
            pl.BlockSpec((1, blk, D), lambda i: (1, i, 0)),
        ],
        out_specs=pl.BlockSpec((blk, D), lambda i: (i, 0)),
        out_shape=jax.ShapeDtypeStruct((N, D), jnp.float32),
    )(partials, partials)


def _sc_body(h_hbm, ei_hbm, ew_hbm, z_hbm, out_hbm,
             acc, rows0, rows1, sstg, dstg, estg,
             gsem0, gsem1, ssem0, ssem1,
             isem0, isem1, isem2, isem3):
    c = lax.axis_index("c")
    s = lax.axis_index("s")
    w_id = s * 2 + c
    count = (NCH - 1 - w_id) // NW + 1
    rows = (rows0, rows1)
    gsems = (gsem0, gsem1)
    ssems = (ssem0, ssem1)
    isems = (isem0, isem1, isem2, isem3)

    pltpu.sync_copy(z_hbm, acc.at[pl.ds(s * ROWS_PER_TILE, ROWS_PER_TILE)])

    def fire_stage(t, slot):
        eb = pl.multiple_of((w_id + t * NW) * CHUNK, CHUNK)
        pltpu.async_copy(ei_hbm.at[pl.ds(eb, CHUNK)], sstg.at[slot],
                         isems[slot])
        pltpu.async_copy(ei_hbm.at[pl.ds(E + eb, CHUNK)], dstg.at[slot],
                         isems[slot])
        pltpu.async_copy(ew_hbm.at[pl.ds(eb, CHUNK)], estg.at[slot],
                         isems[slot])

    def wait_stage(slot):
        pltpu.make_async_copy(ei_hbm.at[pl.ds(0, CHUNK)], sstg.at[slot],
                              isems[slot]).wait()
        pltpu.make_async_copy(ei_hbm.at[pl.ds(0, CHUNK)], dstg.at[slot],
                              isems[slot]).wait()
        pltpu.make_async_copy(ew_hbm.at[pl.ds(0, CHUNK)], estg.at[slot],
                              isems[slot]).wait()

    def fire_gather(stg_slot, row_slot):
        pltpu.async_copy(h_hbm.at[sstg.at[stg_slot]], rows[row_slot],
                         gsems[row_slot])

    def wait_gather(row_slot):
        pltpu.make_async_copy(h_hbm.at[sstg.at[0]], rows[row_slot],
                              gsems[row_slot]).wait()

    def fire_scatter(stg_slot, row_slot):
        pltpu.async_copy(rows[row_slot], acc.at[dstg.at[stg_slot]],
                         ssems[row_slot], add=True)

    def wait_scatter(row_slot):
        pltpu.make_async_copy(rows[row_slot], acc.at[dstg.at[0]],
                              ssems[row_slot]).wait()

    def scale_half(buf, stg_slot, g_lo, g_hi):
        def g_body(g, carry):
            w16 = estg[stg_slot, pl.ds(g * 16, 16)]
            for i in range(16):
                wb = w16[i]
                r = g * 16 + i
                for cc in range(8):
                    buf[r, pl.ds(cc * 16, 16)] = (
                        buf[r, pl.ds(cc * 16, 16)] * wb)
            return carry

        lax.fori_loop(g_lo, g_hi, g_body, 0)

    for t in range(NSTG - 1):
        fire_stage(t, t)
    plsc.subcore_barrier()
    wait_stage(0)
    fire_gather(0, 0)

    def outer(j4, carry):
        t0 = j4 * NSTG
        for b in range(NSTG):
            t = t0 + b
            rb = b % 2
            prb = (b + 1) % 2

            @pl.when(t < count)
            def _():
                wait_gather(rb)
                scale_half(rows[rb], b, 0, 4)

            @pl.when((t >= 1) & (t <= count))
            def _():
                wait_scatter(prb)

            @pl.when(t + NSTG - 1 < count)
            def _():
                fire_stage(t + NSTG - 1, (b + NSTG - 1) % NSTG)

            @pl.when(t + 1 < count)
            def _():
                wait_stage((b + 1) % NSTG)
                fire_gather((b + 1) % NSTG, prb)

            @pl.when(t < count)
            def _():
                scale_half(rows[rb], b, 4, 8)
                fire_scatter(b, rb)
        return carry

    lax.fori_loop(0, MAXC // NSTG, outer, 0)

    plsc.subcore_barrier()
    pltpu.sync_copy(acc.at[pl.ds(s * ROWS_PER_TILE, ROWS_PER_TILE)],
                    out_hbm.at[c, pl.ds(s * ROWS_PER_TILE, ROWS_PER_TILE)])


_sc_aggregate = functools.partial(
    pl.kernel,
    out_type=jax.ShapeDtypeStruct((2, NPAD, D), jnp.float32),
    mesh=plsc.VectorSubcoreMesh(core_axis_name="c", subcore_axis_name="s"),
    compiler_params=pltpu.CompilerParams(needs_layout_passes=False),
    scratch_types=[
        pltpu.VMEM_SHARED((NPAD, D), jnp.float32),
        pltpu.VMEM((CHUNK, D), jnp.float32),
        pltpu.VMEM((CHUNK, D), jnp.float32),
        pltpu.VMEM((NSTG, CHUNK), jnp.int32),
        pltpu.VMEM((NSTG, CHUNK), jnp.int32),
        pltpu.VMEM((NSTG, CHUNK), jnp.float32),
    ] + [pltpu.SemaphoreType.DMA] * 8,
)(_sc_body)


def kernel(x, edge_index, edge_weight, W, b):
    h = _linear(x, W, b)
    zeros = jnp.zeros((ROWS_PER_TILE, D), jnp.float32)
    partials = _sc_aggregate(h, edge_index.reshape(2 * E), edge_weight, zeros)
    return _combine(partials)


# speedup vs baseline: 1.1490x; 1.0098x over previous
import functools

import jax
import jax.numpy as jnp
from jax import lax
from jax.experimental import pallas as pl
from jax.experimental.pallas import tpu as pltpu
from jax.experimental.pallas import tpu_sc as plsc

N = 10000
E = 320000
D = 128

CHUNK = 128
NW = 32
NCH = E // CHUNK
MAXC = 80
NPAD = 10240
ROWS_PER_TILE = NPAD // 16
NSTG = 4


def _mm_body(x_ref, w_ref, b_ref, o_ref):
    o_ref[...] = lax.dot_general(
        x_ref[...], w_ref[...],
        dimension_numbers=(((1,), (1,)), ((), ())),
        preferred_element_type=jnp.float32,
    ) + b_ref[...]


def _linear(x, W, b):
    grid = 10
    blk = N // grid
    return pl.pallas_call(
        _mm_body,
        grid=(grid,),
        in_specs=[
            pl.BlockSpec((blk, D), lambda i: (i, 0)),
            pl.BlockSpec((D, D), lambda i: (0, 0)),
            pl.BlockSpec((1, D), lambda i: (0, 0)),
        ],
        out_specs=pl.BlockSpec((blk, D), lambda i: (i, 0)),
        out_shape=jax.ShapeDtypeStruct((N, D), jnp.float32),
    )(x, W, b.reshape(1, D))


def _add3_body(a_ref, b_ref, o_ref):
    o_ref[...] = a_ref[0] + b_ref[0]


def _combine(partials):
    grid = 10
    blk = N // grid
    return pl.pallas_call(
        _add3_body,
        grid=(grid,),
        in_specs=[
            pl.BlockSpec((1, blk, D), lambda i: (0, i, 0)),
            pl.BlockSpec((1, blk, D), lambda i: (1, i, 0)),
        ],
        out_specs=pl.BlockSpec((blk, D), lambda i: (i, 0)),
        out_shape=jax.ShapeDtypeStruct((N, D), jnp.float32),
    )(partials, partials)


def _sc_body(h_hbm, ei_hbm, ew_hbm, z_hbm, out_hbm,
             acc, rows0, rows1, sstg, dstg, estg,
             gsem0, gsem1, ssem0, ssem1,
             isem0, isem1, isem2, isem3):
    c = lax.axis_index("c")
    s = lax.axis_index("s")
    w_id = s * 2 + c
    count = (NCH - 1 - w_id) // NW + 1
    rows = (rows0, rows1)
    gsems = (gsem0, gsem1)
    ssems = (ssem0, ssem1)
    isems = (isem0, isem1, isem2, isem3)

    def fire_stage(t, slot):
        eb = pl.multiple_of((w_id + t * NW) * CHUNK, CHUNK)
        pltpu.async_copy(ei_hbm.at[pl.ds(eb, CHUNK)], sstg.at[slot],
                         isems[slot])
        pltpu.async_copy(ei_hbm.at[pl.ds(E + eb, CHUNK)], dstg.at[slot],
                         isems[slot])
        pltpu.async_copy(ew_hbm.at[pl.ds(eb, CHUNK)], estg.at[slot],
                         isems[slot])

    def wait_stage(slot):
        pltpu.make_async_copy(ei_hbm.at[pl.ds(0, CHUNK)], sstg.at[slot],
                              isems[slot]).wait()
        pltpu.make_async_copy(ei_hbm.at[pl.ds(0, CHUNK)], dstg.at[slot],
                              isems[slot]).wait()
        pltpu.make_async_copy(ew_hbm.at[pl.ds(0, CHUNK)], estg.at[slot],
                              isems[slot]).wait()

    def fire_gather(stg_slot, row_slot):
        pltpu.async_copy(h_hbm.at[sstg.at[stg_slot]], rows[row_slot],
                         gsems[row_slot])

    def wait_gather(row_slot):
        pltpu.make_async_copy(h_hbm.at[sstg.at[0]], rows[row_slot],
                              gsems[row_slot]).wait()

    def fire_scatter(stg_slot, row_slot):
        pltpu.async_copy(rows[row_slot], acc.at[dstg.at[stg_slot]],
                         ssems[row_slot], add=True)

    def wait_scatter(row_slot):
        pltpu.make_async_copy(rows[row_slot], acc.at[dstg.at[0]],
                              ssems[row_slot]).wait()

    def scale_half(buf, stg_slot, g_lo, g_hi):
        def g_body(g, carry):
            w16 = estg[stg_slot, pl.ds(g * 16, 16)]
            for i in range(16):
                wb = w16[i]
                r = g * 16 + i
                for cc in range(8):
                    buf[r, pl.ds(cc * 16, 16)] = (
                        buf[r, pl.ds(cc * 16, 16)] * wb)
            return carry

        lax.fori_loop(g_lo, g_hi, g_body, 0)

    for t in range(NSTG - 1):
        fire_stage(t, t)
    pltpu.sync_copy(z_hbm, acc.at[pl.ds(s * ROWS_PER_TILE, ROWS_PER_TILE)])
    wait_stage(0)
    fire_gather(0, 0)
    plsc.subcore_barrier()

    def outer(j4, carry):
        t0 = j4 * NSTG
        for b in range(NSTG):
            t = t0 + b
            rb = b % 2
            prb = (b + 1) % 2

            @pl.when(t < count)
            def _():
                wait_gather(rb)
                scale_half(rows[rb], b, 0, 4)

            @pl.when((t >= 1) & (t <= count))
            def _():
                wait_scatter(prb)

            @pl.when(t + NSTG - 1 < count)
            def _():
                fire_stage(t + NSTG - 1, (b + NSTG - 1) % NSTG)

            @pl.when(t + 1 < count)
            def _():
                wait_stage((b + 1) % NSTG)
                fire_gather((b + 1) % NSTG, prb)

            @pl.when(t < count)
            def _():
                scale_half(rows[rb], b, 4, 8)
                fire_scatter(b, rb)
        return carry

    lax.fori_loop(0, MAXC // NSTG, outer, 0)

    plsc.subcore_barrier()
    pltpu.sync_copy(acc.at[pl.ds(s * ROWS_PER_TILE, ROWS_PER_TILE)],
                    out_hbm.at[c, pl.ds(s * ROWS_PER_TILE, ROWS_PER_TILE)])


_sc_aggregate = functools.partial(
    pl.kernel,
    out_type=jax.ShapeDtypeStruct((2, NPAD, D), jnp.float32),
    mesh=plsc.VectorSubcoreMesh(core_axis_name="c", subcore_axis_name="s"),
    compiler_params=pltpu.CompilerParams(needs_layout_passes=False),
    scratch_types=[
        pltpu.VMEM_SHARED((NPAD, D), jnp.float32),
        pltpu.VMEM((CHUNK, D), jnp.float32),
        pltpu.VMEM((CHUNK, D), jnp.float32),
        pltpu.VMEM((NSTG, CHUNK), jnp.int32),
        pltpu.VMEM((NSTG, CHUNK), jnp.int32),
        pltpu.VMEM((NSTG, CHUNK), jnp.float32),
    ] + [pltpu.SemaphoreType.DMA] * 8,
)(_sc_body)


def kernel(x, edge_index, edge_weight, W, b):
    h = _linear(x, W, b)
    zeros = jnp.zeros((ROWS_PER_TILE, D), jnp.float32)
    partials = _sc_aggregate(h, edge_index.reshape(2 * E), edge_weight, zeros)
    return _combine(partials)
